# baseline (speedup 1.0000x reference)
import functools

import jax
import jax.numpy as jnp
from jax.experimental import pallas as pl

N = 10000
F = 128
BM = 400


def _lin_kernel(x_ref, w_ref, b_ref, o_ref):
    o_ref[:] = (
        jnp.dot(x_ref[:], w_ref[:], preferred_element_type=jnp.float32) + b_ref[:]
    )


def _elu(v):
    return jnp.where(v > 0, v, jnp.exp(jnp.minimum(v, 0.0)) - 1.0)


def _layer1_kernel(a_ref, h_ref, o_ref):
    acc = jnp.dot(a_ref[:], h_ref[:], preferred_element_type=jnp.float32)
    o_ref[:] = _elu(acc)


def _layer2_kernel(a_ref, h_ref, w_ref, b_ref, o_ref):
    acc = jnp.dot(a_ref[:], h_ref[:], preferred_element_type=jnp.float32)
    t = _elu(acc)
    o_ref[:] = jnp.dot(t, w_ref[:], preferred_element_type=jnp.float32) + b_ref[:]


@jax.jit
def kernel(x, adjs, W1, b1, Wout, bout):
    W1t = W1.T
    Woutt = Wout.T
    b1r = b1.reshape(1, F)
    boutr = bout.reshape(1, F)

    h0 = pl.pallas_call(
        _lin_kernel,
        out_shape=jax.ShapeDtypeStruct((N, F), jnp.float32),
    )(x, W1t, b1r)

    grid = (N // BM,)
    a_spec = pl.BlockSpec((BM, N), lambda i: (i, 0))
    h_spec = pl.BlockSpec((N, F), lambda i: (0, 0))
    o_spec = pl.BlockSpec((BM, F), lambda i: (i, 0))
    w_spec = pl.BlockSpec((F, F), lambda i: (0, 0))
    b_spec = pl.BlockSpec((1, F), lambda i: (0, 0))

    h1 = pl.pallas_call(
        _layer1_kernel,
        grid=grid,
        in_specs=[a_spec, h_spec],
        out_specs=o_spec,
        out_shape=jax.ShapeDtypeStruct((N, F), jnp.float32),
    )(adjs, h0)

    out = pl.pallas_call(
        _layer2_kernel,
        grid=grid,
        in_specs=[a_spec, h_spec, w_spec, b_spec],
        out_specs=o_spec,
        out_shape=jax.ShapeDtypeStruct((N, F), jnp.float32),
    )(adjs, h1, Woutt, boutr)

    return out
